# Initial kernel scaffold; baseline (speedup 1.0000x reference)
import jax
import jax.numpy as jnp
from jax.experimental import pallas as pl


def kernel(x, table):
    raise NotImplementedError("write your pallas kernel here")


# speedup vs baseline: 13.5256x; 13.5256x over previous
import jax
import jax.numpy as jnp
from jax import lax
from jax.experimental import pallas as pl
from jax.experimental.pallas import tpu as pltpu

S = 512
D = 256
MAX_REL = 32
NTAB = 2 * MAX_REL + 1
KPAD = 128
UROWS = 2 * S
IB = 8


def _body(x_ref, tab_ref, o_ref, u8_ref):
    pid = pl.program_id(0)

    @pl.when(pid == 0)
    def _build_u():
        t = lax.broadcasted_iota(jnp.int32, (UROWS, KPAD), 0)
        k = lax.broadcasted_iota(jnp.int32, (UROWS, KPAD), 1)
        for r in range(IB):
            idx = jnp.clip(t + r - (S - 1), -MAX_REL, MAX_REL) + MAX_REL
            onehot = (idx == k).astype(jnp.float32)
            u8_ref[r] = jnp.dot(onehot, tab_ref[...],
                                preferred_element_type=jnp.float32)

    base = pl.multiple_of((S - IB) - IB * pid, 8)
    for rr in range(IB):
        o_ref[rr] = x_ref[...] + u8_ref[(IB - 1) - rr, pl.ds(base, S), :]


@jax.jit
def kernel(x, table):
    x2 = x.reshape(S, D)
    tab = jnp.zeros((KPAD, D), jnp.float32).at[:NTAB].set(table)
    out = pl.pallas_call(
        _body,
        grid=(S // IB,),
        in_specs=[
            pl.BlockSpec((S, D), lambda i: (0, 0)),
            pl.BlockSpec((KPAD, D), lambda i: (0, 0)),
        ],
        out_specs=pl.BlockSpec((IB, S, D), lambda i: (i, 0, 0)),
        out_shape=jax.ShapeDtypeStruct((S, S, D), jnp.float32),
        scratch_shapes=[pltpu.VMEM((IB, UROWS, D), jnp.float32)],
    )(x2, tab)
    return out
